# Initial kernel scaffold; baseline (speedup 1.0000x reference)
import jax
import jax.numpy as jnp
from jax.experimental import pallas as pl


def kernel(inputs, tables):
    raise NotImplementedError("write your pallas kernel here")


# speedup vs baseline: 16.1061x; 16.1061x over previous
import functools

import jax
import jax.numpy as jnp
from jax import lax
from jax.experimental import pallas as pl
from jax.experimental.pallas import tpu as pltpu
from jax.experimental.pallas import tpu_sc as plsc

F = 26
D = 16
NM = 25
VOCAB = 100000
B = 4096
NW = 32
BPW = B // NW
CB = 4
NCH = BPW // CB
CBF = CB * F
NIDX = NM * CBF
RPC = 128
NPAD = 2688
NCOPY = NPAD // RPC


def _ffm_body(tbl, idxs, out, idx_v, comb_v, rows_v, out_v, sem):
  cid = lax.axis_index("c")
  sid = lax.axis_index("s")
  wid = sid * 2 + cid
  base_b = wid * BPW

  zf = jnp.zeros((16,), jnp.float32)
  zi = jnp.zeros((16,), jnp.int32)
  iota = lax.iota(jnp.int32, 16)

  def _xperm(v, idx):
    return lax.gather(
        v,
        idx[:, None],
        dimension_numbers=lax.GatherDimensionNumbers(
            offset_dims=(), collapsed_slice_dims=(0,), start_index_map=(0,)
        ),
        slice_sizes=(1,),
        mode=lax.GatherScatterMode.PROMISE_IN_BOUNDS,
    )

  def _allsum(v):
    for k in (1, 2, 4, 8):
      v = v + _xperm(v, iota ^ k)
    return v

  idx_v[pl.ds(96, 16)] = zi
  for q in range(NIDX + 8, NPAD, 16):
    comb_v[pl.ds(q, 16)] = zi

  def chunk_body(c, vec):
    b0 = base_b + c * CB
    pltpu.sync_copy(idxs.at[pl.ds(b0 * F, CBF)], idx_v.at[pl.ds(0, CBF)])

    for m in range(NM):
      off = m * VOCAB
      moff = m * CBF
      for q in range(0, 112, 16):
        comb_v[pl.ds(moff + q, 16)] = idx_v[pl.ds(q, 16)] + off

    def fire(i, carry):
      src = tbl.at[comb_v.at[pl.ds(i * RPC, RPC)]]
      dst = rows_v.at[pl.ds(i * RPC, RPC), :]
      pltpu.async_copy(src, dst, sem)
      return carry

    lax.fori_loop(0, NCOPY, fire, 0)

    pltpu.make_async_copy(
        tbl.at[pl.ds(0, NPAD), :], rows_v, sem
    ).wait()

    for b in range(CB):
      boff = b * F

      def m_body(m, acc, _boff=boff):
        a_base = m * CBF + _boff
        b_base = _boff + m + 1

        def i_body(i, acc2):
          va = rows_v[a_base + i, :]
          vb = rows_v[i * CBF + b_base, :]
          return acc2 + va * vb

        return lax.fori_loop(0, m + 1, i_body, acc)

      acc = lax.fori_loop(0, NM, m_body, zf)
      tot = _allsum(acc)
      lane = (c % 4) * CB + b
      vec = vec + jnp.where(iota == lane, tot, 0.0)

    @pl.when(c % 4 == 3)
    def _():
      out_v[pl.ds((c // 4) * 16, 16)] = vec

    return jnp.where(c % 4 == 3, zf, vec)

  lax.fori_loop(0, NCH, chunk_body, zf)
  pltpu.sync_copy(out_v, out.at[pl.ds(base_b, BPW)])


def _make_kernel(interpret=False):
  mesh = plsc.VectorSubcoreMesh(
      core_axis_name="c", subcore_axis_name="s", num_cores=2, num_subcores=16
  )
  return pl.kernel(
      _ffm_body,
      out_type=jax.ShapeDtypeStruct((B,), jnp.float32),
      mesh=mesh,
      scratch_types=[
          pltpu.VMEM((112,), jnp.int32),
          pltpu.VMEM((NPAD,), jnp.int32),
          pltpu.VMEM((NPAD, D), jnp.float32),
          pltpu.VMEM((BPW,), jnp.float32),
          pltpu.SemaphoreType.DMA,
      ],
      compiler_params=pltpu.CompilerParams(use_tc_tiling_on_sc=False),
      interpret=interpret,
  )


@jax.jit
def kernel(inputs, tables):
  tbl = tables.reshape(NM * VOCAB, D)
  idxs = inputs.reshape(B * F)
  out = _make_kernel()(tbl, idxs)
  return out.reshape(B, 1)


# speedup vs baseline: 21.5789x; 1.3398x over previous
import functools

import numpy as np
import jax
import jax.numpy as jnp
from jax import lax
from jax.experimental import pallas as pl
from jax.experimental.pallas import tpu as pltpu
from jax.experimental.pallas import tpu_sc as plsc

F = 26
D = 16
NM = 25
VOCAB = 100000
B = 4096
NW = 32
BPW = B // NW
CB = 4
NCH = BPW // CB
CBF = CB * F
NROW = NM * CBF
ZROW = NROW
RHALF = 2704
NPAIR = 325
NGRP = 21
NPAD_PAIR = NGRP * 16


def _pair_tables():
  pa = np.full((NPAD_PAIR,), ZROW, np.int32)
  pb = np.full((NPAD_PAIR,), ZROW, np.int32)
  p = 0
  for m in range(NM):
    for i in range(m + 1):
      pa[p] = m * CBF + i
      pb[p] = i * CBF + (m + 1)
      p += 1
  assert p == NPAIR
  return jnp.asarray(pa), jnp.asarray(pb)


def _ffm_body(tbl, idxs, pa_h, pb_h, out,
              idx_all, rows_v, out_v, pa_v, pb_v, sem0, sem1):
  cid = lax.axis_index("c")
  sid = lax.axis_index("s")
  wid = sid * 2 + cid
  base_b = wid * BPW

  zf = jnp.zeros((16,), jnp.float32)
  iota = lax.iota(jnp.int32, 16)

  pltpu.sync_copy(idxs.at[pl.ds(wid * NCH, NCH), :], idx_all)
  pltpu.sync_copy(pa_h, pa_v)
  pltpu.sync_copy(pb_h, pb_v)
  for h in range(2):
    for r in range(NROW, RHALF):
      rows_v[h * RHALF + r, :] = zf

  def fire(c):
    par = c % 2
    rbase = par * RHALF
    idx_sl = idx_all.at[c]

    def issue(sem):
      def body(m, carry):
        pltpu.async_copy(
            tbl.at[m].at[idx_sl],
            rows_v.at[pl.ds(rbase + m * CBF, CBF), :],
            sem,
        )
        return carry
      lax.fori_loop(0, NM, body, 0)

    @pl.when(par == 0)
    def _():
      issue(sem0)

    @pl.when(par == 1)
    def _():
      issue(sem1)

  def drain(c):
    par = c % 2
    rbase = par * RHALF
    dummy = tbl.at[0].at[pl.ds(0, NROW), :]

    @pl.when(par == 0)
    def _():
      pltpu.make_async_copy(
          dummy, rows_v.at[pl.ds(rbase, NROW), :], sem0).wait()

    @pl.when(par == 1)
    def _():
      pltpu.make_async_copy(
          dummy, rows_v.at[pl.ds(rbase, NROW), :], sem1).wait()

  def _xperm(v, idx):
    return lax.gather(
        v,
        idx[:, None],
        dimension_numbers=lax.GatherDimensionNumbers(
            offset_dims=(), collapsed_slice_dims=(0,), start_index_map=(0,)
        ),
        slice_sizes=(1,),
        mode=lax.GatherScatterMode.PROMISE_IN_BOUNDS,
    )

  def compute(c):
    par = c % 2
    rbase = par * RHALF

    def group(g, accs):
      av = pa_v[pl.ds(g * 16, 16)] + rbase
      bv = pb_v[pl.ds(g * 16, 16)] + rbase
      ar = [av + b * F for b in range(CB)]
      br = [bv + b * F for b in range(CB)]
      accs = list(accs)
      for d in range(D):
        dv = (iota + d) & 15
        for b in range(CB):
          va = plsc.load_gather(rows_v, [ar[b], dv])
          vb = plsc.load_gather(rows_v, [br[b], dv])
          accs[b] = accs[b] + va * vb
      return tuple(accs)

    accs = lax.fori_loop(0, NGRP, group, (zf,) * CB)
    for b in range(CB):
      tot = accs[b]
      for k in (1, 2, 4, 8):
        tot = tot + _xperm(tot, iota ^ k)
      plsc.store_scatter(
          out_v, [jnp.broadcast_to(c * CB + b, (16,))], tot, mask=iota == 0
      )

  fire(0)

  def chunk(c, carry):
    @pl.when(c + 1 < NCH)
    def _():
      fire(c + 1)

    drain(c)
    compute(c)
    return carry

  lax.fori_loop(0, NCH, chunk, 0)
  pltpu.sync_copy(out_v, out.at[pl.ds(base_b, BPW)])


def _make_kernel():
  mesh = plsc.VectorSubcoreMesh(
      core_axis_name="c", subcore_axis_name="s", num_cores=2, num_subcores=16
  )
  return pl.kernel(
      _ffm_body,
      out_type=jax.ShapeDtypeStruct((B,), jnp.float32),
      mesh=mesh,
      scratch_types=[
          pltpu.VMEM((NCH, CBF), jnp.int32),
          pltpu.VMEM((2 * RHALF, D), jnp.float32),
          pltpu.VMEM((BPW,), jnp.float32),
          pltpu.VMEM((NPAD_PAIR,), jnp.int32),
          pltpu.VMEM((NPAD_PAIR,), jnp.int32),
          pltpu.SemaphoreType.DMA,
          pltpu.SemaphoreType.DMA,
      ],
      compiler_params=pltpu.CompilerParams(
          use_tc_tiling_on_sc=False, needs_layout_passes=False
      ),
  )


@jax.jit
def kernel(inputs, tables):
  idxs = inputs.reshape(B // CB, CBF)
  pa, pb = _pair_tables()
  out = _make_kernel()(tables, idxs, pa, pb)
  return out.reshape(B, 1)
